# speedup vs baseline: 324.1308x; 324.1308x over previous
import functools

import jax
import jax.numpy as jnp
from jax import lax
from jax.experimental import pallas as pl
from jax.experimental.pallas import tpu as pltpu
from jax.experimental.pallas import tpu_sc as plsc

NUM_ROWS = 1000
NUM_COLS = 1000
NUM_IDX = 16384 * 200
NC = 2
NS = 16
NW = NC * NS
PER_W = NUM_IDX // NW
CHUNK = 2048
N_CHUNKS = PER_W // CHUNK
LANES = 16


def _combine_body(a_ref, b_ref, c_ref, m_ref, d_ref):
    m = m_ref[0, 0]
    d_ref[...] = m * (a_ref[...] + c_ref[...]) + b_ref[...]


def _combine(a2, b2, c2, m2):
    return pl.pallas_call(
        _combine_body,
        out_shape=jax.ShapeDtypeStruct((NUM_ROWS, NUM_COLS), jnp.float32),
    )(a2, b2, c2, m2)


def _gather_sum_body(idx_hbm, d_hbm, out_hbm, idx_v, vals_v, acc_v, sem):
    wid = lax.axis_index("s") * NC + lax.axis_index("c")
    base = wid * PER_W

    def chunk_body(i, acc):
        off = base + i * CHUNK
        pltpu.sync_copy(idx_hbm.at[pl.ds(off, CHUNK)], idx_v)
        pltpu.async_copy(d_hbm.at[idx_v], vals_v, sem).wait()

        def add_body(j, a):
            return a + vals_v[pl.ds(j * LANES, LANES)]

        return lax.fori_loop(0, CHUNK // LANES, add_body, acc)

    acc = lax.fori_loop(0, N_CHUNKS, chunk_body,
                        jnp.zeros((LANES,), jnp.float32))
    acc_v[...] = acc
    pltpu.sync_copy(acc_v, out_hbm.at[wid])


_gather_sum = pl.kernel(
    _gather_sum_body,
    out_type=jax.ShapeDtypeStruct((NW, LANES), jnp.float32),
    mesh=plsc.VectorSubcoreMesh(core_axis_name="c", subcore_axis_name="s"),
    scratch_types=[
        pltpu.VMEM((CHUNK,), jnp.int32),
        pltpu.VMEM((CHUNK,), jnp.float32),
        pltpu.VMEM((LANES,), jnp.float32),
        pltpu.SemaphoreType.DMA,
    ],
)


def kernel(c_t_indices, M_prev, A, B, C):
    a2 = A.reshape(NUM_ROWS, NUM_COLS)
    b2 = B.reshape(NUM_ROWS, NUM_COLS)
    c2 = C.reshape(NUM_ROWS, NUM_COLS)
    d_flat = _combine(a2, b2, c2, M_prev.reshape(1, 1)).reshape(-1)
    idx_flat = c_t_indices.reshape(-1)
    partials = _gather_sum(idx_flat, d_flat)
    return jnp.sum(partials).reshape(1)


# speedup vs baseline: 487.1206x; 1.5029x over previous
import functools

import jax
import jax.numpy as jnp
from jax import lax
from jax.experimental import pallas as pl
from jax.experimental.pallas import tpu as pltpu
from jax.experimental.pallas import tpu_sc as plsc

NUM_SAMP = 1000000
NUM_IDX = 16384 * 200
NC = 2
NS = 16
NW = NC * NS
PER_W = NUM_IDX // NW
CHUNK = 2048
N_CHUNKS = PER_W // CHUNK
NBUF = 2
LANES = 16
CBLK = 32768


def _combine_body(m_ref, a_ref, b_ref, c_ref, d_ref):
    m = m_ref[0]
    d_ref[...] = m * (a_ref[...] + c_ref[...]) + b_ref[...]


def _combine(a, b, c, m):
    blk = pl.BlockSpec((CBLK,), lambda i: (i,))
    return pl.pallas_call(
        _combine_body,
        grid=(pl.cdiv(NUM_SAMP, CBLK),),
        in_specs=[pl.BlockSpec(memory_space=pltpu.SMEM), blk, blk, blk],
        out_specs=blk,
        out_shape=jax.ShapeDtypeStruct((NUM_SAMP,), jnp.float32),
    )(m, a, b, c)


def _gather_sum_body(idx_hbm, d_hbm, out_hbm,
                     idx0, idx1, vals0, vals1, acc_v, sem0, sem1):
    wid = lax.axis_index("s") * NC + lax.axis_index("c")
    base = wid * PER_W
    idx_bufs = (idx0, idx1)
    val_bufs = (vals0, vals1)
    sems = (sem0, sem1)

    for b in range(NBUF):
        pltpu.sync_copy(idx_hbm.at[pl.ds(base + b * CHUNK, CHUNK)],
                        idx_bufs[b])
        pltpu.async_copy(d_hbm.at[idx_bufs[b]], val_bufs[b], sems[b])

    def group_body(g, acc):
        for b in range(NBUF):
            i = g * NBUF + b
            pltpu.make_async_copy(d_hbm.at[idx_bufs[b]], val_bufs[b],
                                  sems[b]).wait()

            def add_body(j, a, _v=val_bufs[b]):
                u = _v[pl.ds(j * 32, LANES)] + _v[pl.ds(j * 32 + LANES, LANES)]
                return a + u

            acc = lax.fori_loop(0, CHUNK // 32, add_body, acc, unroll=4)

            nxt = i + NBUF

            @pl.when(nxt < N_CHUNKS)
            def _(b=b, nxt=nxt):
                pltpu.sync_copy(idx_hbm.at[pl.ds(base + nxt * CHUNK, CHUNK)],
                                idx_bufs[b])
                pltpu.async_copy(d_hbm.at[idx_bufs[b]], val_bufs[b], sems[b])

        return acc

    acc = lax.fori_loop(0, N_CHUNKS // NBUF, group_body,
                        jnp.zeros((LANES,), jnp.float32))
    acc_v[...] = acc
    pltpu.sync_copy(acc_v, out_hbm.at[wid])


_gather_sum = pl.kernel(
    _gather_sum_body,
    out_type=jax.ShapeDtypeStruct((NW, LANES), jnp.float32),
    mesh=plsc.VectorSubcoreMesh(core_axis_name="c", subcore_axis_name="s"),
    scratch_types=[
        pltpu.VMEM((CHUNK,), jnp.int32),
        pltpu.VMEM((CHUNK,), jnp.int32),
        pltpu.VMEM((CHUNK,), jnp.float32),
        pltpu.VMEM((CHUNK,), jnp.float32),
        pltpu.VMEM((LANES,), jnp.float32),
        pltpu.SemaphoreType.DMA,
        pltpu.SemaphoreType.DMA,
    ],
)


def kernel(c_t_indices, M_prev, A, B, C):
    d_flat = _combine(A, B, C, M_prev)
    idx_flat = c_t_indices.reshape(-1)
    partials = _gather_sum(idx_flat, d_flat)
    return jnp.sum(partials).reshape(1)
